# baseline (speedup 1.0000x reference)
import jax
import jax.numpy as jnp
from jax import lax
from jax.experimental import pallas as pl
from jax.experimental.pallas import tpu as pltpu
from jax.experimental.pallas import tpu_sc as plsc

B = 64
N = 262144
ROWS = 512
NW = 32
ROWS_PER_W = B // NW
CROWS = 16
CHUNK = CROWS * 512
NCHUNK = N // CHUNK
NV = CHUNK // 16
UNROLL = 8

POS_LO = 2621.43
POS_UP = 259521.58
T_LO = 2621
T_UP = 259521

_INT_MIN_PY = -2147483648


def _sc_body(x_hbm, out_hbm, hist, buf0, buf1, outv, sem0, sem1):
    _INT_MIN = jnp.int32(_INT_MIN_PY)
    cid = lax.axis_index("c")
    sid = lax.axis_index("s")
    wid = sid * 2 + cid
    iota = lax.iota(jnp.int32, 16)
    zeros16 = jnp.zeros((16,), jnp.int32)
    ones16 = jnp.ones((16,), jnp.int32)

    def start_chunk(row, c, buf, sem):
        pltpu.make_async_copy(
            x_hbm.at[row, pl.ds(c * CROWS, CROWS)], buf, sem).start()

    def wait_chunk(row, c, buf, sem):
        pltpu.make_async_copy(
            x_hbm.at[row, pl.ds(c * CROWS, CROWS)], buf, sem).wait()

    def process(buf):
        def _hist_body(j, _):
            j0 = j * UNROLL
            for u in range(UNROLL):
                jj = j0 + u
                ri = lax.shift_right_logical(jj, 5)
                col = lax.shift_left(jnp.bitwise_and(jj, 31), 4)
                v = buf[ri, pl.ds(col, 16)]
                k32 = plsc.bitcast(v, jnp.int32)
                m = lax.shift_right_arithmetic(k32, 31)
                key = lax.bitwise_xor(k32, lax.bitwise_or(m, _INT_MIN))
                lo4 = jnp.bitwise_and(lax.shift_right_logical(key, 16), 15)
                addr = jnp.bitwise_or(lax.shift_left(lo4, 12),
                                      lax.shift_right_logical(key, 20))
                plsc.addupdate_scatter(hist, [addr], ones16)
            return 0

        lax.fori_loop(0, NV // UNROLL, _hist_body, 0)

    for rr in range(ROWS_PER_W):
        row = wid * ROWS_PER_W + rr

        def _zero_hist(i, _):
            base = i * 256
            for u in range(16):
                hist[pl.ds(base + u * 16, 16)] = zeros16
            return 0

        lax.fori_loop(0, 256, _zero_hist, 0)

        start_chunk(row, 0, buf0, sem0)
        def _chunk_body(k, _):
            c0 = k * 2
            wait_chunk(row, c0, buf0, sem0)
            start_chunk(row, c0 + 1, buf1, sem1)
            process(buf0)
            wait_chunk(row, c0 + 1, buf1, sem1)

            @pl.when(k < NCHUNK // 2 - 1)
            def _():
                start_chunk(row, c0 + 2, buf0, sem0)

            process(buf1)
            return 0

        lax.fori_loop(0, NCHUNK // 2, _chunk_body, 0)

        def _plane_sums(r):
            base = r * 16
            acc = hist[pl.ds(base, 16)]
            for l in range(1, 16):
                acc = acc + hist[pl.ds(l * 4096 + base, 16)]
            return acc

        def _cscan(r, carry):
            cum, rL, cumL, rU, cumU = carry
            tot = jnp.sum(_plane_sums(r))
            nxt = cum + tot
            condL = (cum <= T_LO) & (T_LO < nxt)
            condU = (cum <= T_UP) & (T_UP < nxt)
            rL = lax.select(condL, r, rL)
            cumL = lax.select(condL, cum, cumL)
            rU = lax.select(condU, r, rU)
            cumU = lax.select(condU, cum, cumU)
            return nxt, rL, cumL, rU, cumU

        z = jnp.int32(0)
        _, rL, cumL, rU, cumU = lax.fori_loop(0, 256, _cscan, (z, z, z, z, z))

        def _locate(r, cumbef, t):
            acc = _plane_sums(r)
            s = plsc.cumsum(acc)
            lane = jnp.sum((cumbef + s <= t).astype(jnp.int32))
            cbef_g = cumbef + jnp.sum(jnp.where(iota == lane, s - acc, 0))
            grp = r * 16 + lane
            vec = plsc.load_gather(hist, [iota * 4096 + grp])
            s2 = plsc.cumsum(vec)
            lane2 = jnp.sum((cbef_g + s2 <= t).astype(jnp.int32))
            cnt = jnp.sum(jnp.where(iota == lane2, vec, 0))
            cbef = cbef_g + jnp.sum(jnp.where(iota == lane2, s2 - vec, 0))
            fbin = grp * 16 + lane2
            klo = lax.shift_left(fbin, 16)
            khi = lax.shift_left(fbin + 1, 16)
            kv = jnp.where(iota == 0, klo, khi)
            bits = jnp.where(kv < 0,
                             lax.bitwise_xor(kv, _INT_MIN),
                             lax.bitwise_not(kv))
            fv = plsc.bitcast(bits, jnp.float32)
            neg_big = jnp.float32(-3.4e38)
            vlo = jnp.max(jnp.where(iota == 0, fv, neg_big))
            vhi = jnp.max(jnp.where(iota == 1, fv, neg_big))
            return vlo, vhi, cbef.astype(jnp.float32), cnt.astype(jnp.float32)

        vloL, vhiL, cbefL, cntL = _locate(rL, cumL, T_LO)
        vloU, vhiU, cbefU, cntU = _locate(rU, cumU, T_UP)
        fz = jnp.float32(0.0)
        resv = fz
        for lane_ix, val in ((0, vloL), (1, vhiL), (2, cbefL), (3, cntL),
                             (4, vloU), (5, vhiU), (6, cbefU), (7, cntU)):
            resv = jnp.where(iota == lane_ix, val, resv)
        outv[0, :] = resv
        pltpu.sync_copy(outv, out_hbm.at[row])


_sc_quantile = pl.kernel(
    _sc_body,
    out_type=jax.ShapeDtypeStruct((B, 1, 16), jnp.float32),
    mesh=plsc.VectorSubcoreMesh(core_axis_name="c", subcore_axis_name="s"),
    compiler_params=pltpu.CompilerParams(needs_layout_passes=False),
    scratch_types=[
        pltpu.VMEM((65536,), jnp.int32),
        pltpu.VMEM((CROWS, 512), jnp.float32),
        pltpu.VMEM((CROWS, 512), jnp.float32),
        pltpu.VMEM((1, 16), jnp.float32),
        pltpu.SemaphoreType.DMA,
        pltpu.SemaphoreType.DMA,
    ],
)


def _norm_body(b_ref, x_ref, o_ref):
    def interp(vlo, vhi, cbef, cnt, pos):
        return vlo + (vhi - vlo) * ((jnp.float32(pos) - cbef
                                     + jnp.float32(0.5)) / cnt)

    lo = interp(b_ref[0, 0, 0], b_ref[0, 0, 1], b_ref[0, 0, 2],
                b_ref[0, 0, 3], POS_LO)
    up = interp(b_ref[0, 0, 4], b_ref[0, 0, 5], b_ref[0, 0, 6],
                b_ref[0, 0, 7], POS_UP)
    xv = x_ref[...]
    o_ref[...] = (jnp.maximum(jnp.minimum(xv, up), lo) - lo) / (up - lo)


_tc_normalize = pl.pallas_call(
    _norm_body,
    grid=(B,),
    in_specs=[
        pl.BlockSpec((1, 1, 16), lambda i: (i, 0, 0)),
        pl.BlockSpec((1, 1, 512, 512), lambda i: (i, 0, 0, 0)),
    ],
    out_specs=pl.BlockSpec((1, 1, 512, 512), lambda i: (i, 0, 0, 0)),
    out_shape=jax.ShapeDtypeStruct((B, 1, 512, 512), jnp.float32),
)


def kernel(x):
    xf = x.reshape(B, ROWS, 512)
    bounds = _sc_quantile(xf)
    return _tc_normalize(bounds, x)


# speedup vs baseline: 2.4193x; 2.4193x over previous
import jax
import jax.numpy as jnp
from jax import lax
from jax.experimental import pallas as pl
from jax.experimental.pallas import tpu as pltpu
from jax.experimental.pallas import tpu_sc as plsc

B = 64
N = 262144
ROWS = 512
NW = 32
ROWS_PER_W = B // NW
CROWS = 16
CHUNK = CROWS * 512
NCHUNK = N // CHUNK
NV = CHUNK // 16
UNROLL = 8

POS_LO = 2621.43
POS_UP = 259521.58
T_LO = 2621
T_UP = 259521

_INT_MIN_PY = -2147483648


def _sc_body(x_hbm, out_hbm, hist, buf0, buf1, outv, sem0, sem1):
    _INT_MIN = jnp.int32(_INT_MIN_PY)
    cid = lax.axis_index("c")
    sid = lax.axis_index("s")
    wid = sid * 2 + cid
    iota = lax.iota(jnp.int32, 16)
    zeros16 = jnp.zeros((16,), jnp.int32)
    ones16 = jnp.ones((16,), jnp.int32)

    def start_chunk(row, c, buf, sem):
        pltpu.make_async_copy(
            x_hbm.at[row, pl.ds(c * CROWS, CROWS)], buf, sem).start()

    def wait_chunk(row, c, buf, sem):
        pltpu.make_async_copy(
            x_hbm.at[row, pl.ds(c * CROWS, CROWS)], buf, sem).wait()

    def process(buf):
        def _hist_body(j, _):
            j0 = j * UNROLL
            vs = []
            for u in range(UNROLL):
                jj = j0 + u
                ri = lax.shift_right_logical(jj, 5)
                col = lax.shift_left(jnp.bitwise_and(jj, 31), 4)
                vs.append(buf[ri, pl.ds(col, 16)])
            addrs = []
            for v in vs:
                k32 = plsc.bitcast(v, jnp.int32)
                m = lax.shift_right_arithmetic(k32, 31)
                key = lax.bitwise_xor(k32, lax.bitwise_or(m, _INT_MIN))
                lo4 = jnp.bitwise_and(lax.shift_right_logical(key, 16), 15)
                addrs.append(jnp.bitwise_or(lax.shift_left(lo4, 12),
                                            lax.shift_right_logical(key, 20)))
            for addr in addrs:
                plsc.addupdate_scatter(hist, [addr], ones16)
            return 0

        lax.fori_loop(0, NV // UNROLL, _hist_body, 0)

    for rr in range(ROWS_PER_W):
        row = wid * ROWS_PER_W + rr

        def _zero_hist(i, _):
            base = i * 256
            for u in range(16):
                hist[pl.ds(base + u * 16, 16)] = zeros16
            return 0

        lax.fori_loop(0, 256, _zero_hist, 0)

        start_chunk(row, 0, buf0, sem0)
        def _chunk_body(k, _):
            c0 = k * 2
            wait_chunk(row, c0, buf0, sem0)
            start_chunk(row, c0 + 1, buf1, sem1)
            process(buf0)
            wait_chunk(row, c0 + 1, buf1, sem1)

            @pl.when(k < NCHUNK // 2 - 1)
            def _():
                start_chunk(row, c0 + 2, buf0, sem0)

            process(buf1)
            return 0

        lax.fori_loop(0, NCHUNK // 2, _chunk_body, 0)

        def _plane_sums(r):
            base = r * 16
            acc = hist[pl.ds(base, 16)]
            for l in range(1, 16):
                acc = acc + hist[pl.ds(l * 4096 + base, 16)]
            return acc

        def _cscan(r, carry):
            cum, rL, cumL, rU, cumU = carry
            tot = jnp.sum(_plane_sums(r))
            nxt = cum + tot
            condL = (cum <= T_LO) & (T_LO < nxt)
            condU = (cum <= T_UP) & (T_UP < nxt)
            rL = lax.select(condL, r, rL)
            cumL = lax.select(condL, cum, cumL)
            rU = lax.select(condU, r, rU)
            cumU = lax.select(condU, cum, cumU)
            return nxt, rL, cumL, rU, cumU

        z = jnp.int32(0)
        _, rL, cumL, rU, cumU = lax.fori_loop(0, 256, _cscan, (z, z, z, z, z))

        def _locate(r, cumbef, t):
            acc = _plane_sums(r)
            s = plsc.cumsum(acc)
            lane = jnp.sum((cumbef + s <= t).astype(jnp.int32))
            cbef_g = cumbef + jnp.sum(jnp.where(iota == lane, s - acc, 0))
            grp = r * 16 + lane
            vec = plsc.load_gather(hist, [iota * 4096 + grp])
            s2 = plsc.cumsum(vec)
            lane2 = jnp.sum((cbef_g + s2 <= t).astype(jnp.int32))
            cnt = jnp.sum(jnp.where(iota == lane2, vec, 0))
            cbef = cbef_g + jnp.sum(jnp.where(iota == lane2, s2 - vec, 0))
            fbin = grp * 16 + lane2
            klo = lax.shift_left(fbin, 16)
            khi = lax.shift_left(fbin + 1, 16)
            kv = jnp.where(iota == 0, klo, khi)
            bits = jnp.where(kv < 0,
                             lax.bitwise_xor(kv, _INT_MIN),
                             lax.bitwise_not(kv))
            fv = plsc.bitcast(bits, jnp.float32)
            neg_big = jnp.float32(-3.4e38)
            vlo = jnp.max(jnp.where(iota == 0, fv, neg_big))
            vhi = jnp.max(jnp.where(iota == 1, fv, neg_big))
            return vlo, vhi, cbef.astype(jnp.float32), cnt.astype(jnp.float32)

        vloL, vhiL, cbefL, cntL = _locate(rL, cumL, T_LO)
        vloU, vhiU, cbefU, cntU = _locate(rU, cumU, T_UP)
        fz = jnp.float32(0.0)
        resv = fz
        for lane_ix, val in ((0, vloL), (1, vhiL), (2, cbefL), (3, cntL),
                             (4, vloU), (5, vhiU), (6, cbefU), (7, cntU)):
            resv = jnp.where(iota == lane_ix, val, resv)
        outv[0, :] = resv
        pltpu.sync_copy(outv, out_hbm.at[row])


_sc_quantile = pl.kernel(
    _sc_body,
    out_type=jax.ShapeDtypeStruct((B, 1, 16), jnp.float32),
    mesh=plsc.VectorSubcoreMesh(core_axis_name="c", subcore_axis_name="s"),
    compiler_params=pltpu.CompilerParams(needs_layout_passes=False),
    scratch_types=[
        pltpu.VMEM((65536,), jnp.int32),
        pltpu.VMEM((CROWS, 512), jnp.float32),
        pltpu.VMEM((CROWS, 512), jnp.float32),
        pltpu.VMEM((1, 16), jnp.float32),
        pltpu.SemaphoreType.DMA,
        pltpu.SemaphoreType.DMA,
    ],
)


def _norm_body(b_ref, x_ref, o_ref):
    def interp(vlo, vhi, cbef, cnt, pos):
        return vlo + (vhi - vlo) * ((jnp.float32(pos) - cbef
                                     + jnp.float32(0.5)) / cnt)

    lo = interp(b_ref[0, 0, 0], b_ref[0, 0, 1], b_ref[0, 0, 2],
                b_ref[0, 0, 3], POS_LO)
    up = interp(b_ref[0, 0, 4], b_ref[0, 0, 5], b_ref[0, 0, 6],
                b_ref[0, 0, 7], POS_UP)
    xv = x_ref[...]
    o_ref[...] = (jnp.maximum(jnp.minimum(xv, up), lo) - lo) / (up - lo)


_tc_normalize = pl.pallas_call(
    _norm_body,
    grid=(B,),
    in_specs=[
        pl.BlockSpec((1, 1, 16), lambda i: (i, 0, 0)),
        pl.BlockSpec((1, 1, 512, 512), lambda i: (i, 0, 0, 0)),
    ],
    out_specs=pl.BlockSpec((1, 1, 512, 512), lambda i: (i, 0, 0, 0)),
    out_shape=jax.ShapeDtypeStruct((B, 1, 512, 512), jnp.float32),
)


def kernel(x):
    xf = x.reshape(B, ROWS, 512)
    bounds = _sc_quantile(xf)
    return _tc_normalize(bounds, x)


# speedup vs baseline: 2.9435x; 1.2167x over previous
import jax
import jax.numpy as jnp
from jax import lax
from jax.experimental import pallas as pl
from jax.experimental.pallas import tpu as pltpu
from jax.experimental.pallas import tpu_sc as plsc

B = 64
N = 262144
ROWS = 512
NW = 32
ROWS_PER_W = B // NW
CROWS = 16
CHUNK = CROWS * 512
NCHUNK = N // CHUNK
NV = CHUNK // 16
UNROLL = 16

POS_LO = 2621.43
POS_UP = 259521.58
T_LO = 2621
T_UP = 259521

_INT_MIN_PY = -2147483648


def _sc_body(x_hbm, out_hbm, hist, buf0, buf1, outv, sem0, sem1):
    _INT_MIN = jnp.int32(_INT_MIN_PY)
    cid = lax.axis_index("c")
    sid = lax.axis_index("s")
    wid = sid * 2 + cid
    iota = lax.iota(jnp.int32, 16)
    zeros16 = jnp.zeros((16,), jnp.int32)
    ones16 = jnp.ones((16,), jnp.int32)

    def start_chunk(row, c, buf, sem):
        pltpu.make_async_copy(
            x_hbm.at[row, pl.ds(c * CROWS, CROWS)], buf, sem).start()

    def wait_chunk(row, c, buf, sem):
        pltpu.make_async_copy(
            x_hbm.at[row, pl.ds(c * CROWS, CROWS)], buf, sem).wait()

    def process(buf):
        def _hist_body(j, _):
            j0 = j * UNROLL
            vs = []
            for u in range(UNROLL):
                jj = j0 + u
                ri = lax.shift_right_logical(jj, 5)
                col = lax.shift_left(jnp.bitwise_and(jj, 31), 4)
                vs.append(buf[ri, pl.ds(col, 16)])
            addrs = []
            for v in vs:
                k32 = plsc.bitcast(v, jnp.int32)
                m = lax.shift_right_arithmetic(k32, 31)
                key = lax.bitwise_xor(k32, lax.bitwise_or(m, _INT_MIN))
                lo4 = jnp.bitwise_and(lax.shift_right_logical(key, 16), 15)
                addrs.append(jnp.bitwise_or(lax.shift_left(lo4, 12),
                                            lax.shift_right_logical(key, 20)))
            for addr in addrs:
                plsc.addupdate_scatter(hist, [addr], ones16)
            return 0

        lax.fori_loop(0, NV // UNROLL, _hist_body, 0)

    for rr in range(ROWS_PER_W):
        row = wid * ROWS_PER_W + rr

        def _zero_hist(i, _):
            base = i * 256
            for u in range(16):
                hist[pl.ds(base + u * 16, 16)] = zeros16
            return 0

        lax.fori_loop(0, 256, _zero_hist, 0)

        start_chunk(row, 0, buf0, sem0)
        def _chunk_body(k, _):
            c0 = k * 2
            wait_chunk(row, c0, buf0, sem0)
            start_chunk(row, c0 + 1, buf1, sem1)
            process(buf0)
            wait_chunk(row, c0 + 1, buf1, sem1)

            @pl.when(k < NCHUNK // 2 - 1)
            def _():
                start_chunk(row, c0 + 2, buf0, sem0)

            process(buf1)
            return 0

        lax.fori_loop(0, NCHUNK // 2, _chunk_body, 0)

        def _plane_sums(r):
            base = r * 16
            acc = hist[pl.ds(base, 16)]
            for l in range(1, 16):
                acc = acc + hist[pl.ds(l * 4096 + base, 16)]
            return acc

        def _cscan(r, carry):
            cum, rL, cumL, rU, cumU = carry
            tot = jnp.sum(_plane_sums(r))
            nxt = cum + tot
            condL = (cum <= T_LO) & (T_LO < nxt)
            condU = (cum <= T_UP) & (T_UP < nxt)
            rL = lax.select(condL, r, rL)
            cumL = lax.select(condL, cum, cumL)
            rU = lax.select(condU, r, rU)
            cumU = lax.select(condU, cum, cumU)
            return nxt, rL, cumL, rU, cumU

        z = jnp.int32(0)
        _, rL, cumL, rU, cumU = lax.fori_loop(0, 256, _cscan, (z, z, z, z, z))

        def _locate(r, cumbef, t):
            acc = _plane_sums(r)
            s = plsc.cumsum(acc)
            lane = jnp.sum((cumbef + s <= t).astype(jnp.int32))
            cbef_g = cumbef + jnp.sum(jnp.where(iota == lane, s - acc, 0))
            grp = r * 16 + lane
            vec = plsc.load_gather(hist, [iota * 4096 + grp])
            s2 = plsc.cumsum(vec)
            lane2 = jnp.sum((cbef_g + s2 <= t).astype(jnp.int32))
            cnt = jnp.sum(jnp.where(iota == lane2, vec, 0))
            cbef = cbef_g + jnp.sum(jnp.where(iota == lane2, s2 - vec, 0))
            fbin = grp * 16 + lane2
            klo = lax.shift_left(fbin, 16)
            khi = lax.shift_left(fbin + 1, 16)
            kv = jnp.where(iota == 0, klo, khi)
            bits = jnp.where(kv < 0,
                             lax.bitwise_xor(kv, _INT_MIN),
                             lax.bitwise_not(kv))
            fv = plsc.bitcast(bits, jnp.float32)
            neg_big = jnp.float32(-3.4e38)
            vlo = jnp.max(jnp.where(iota == 0, fv, neg_big))
            vhi = jnp.max(jnp.where(iota == 1, fv, neg_big))
            return vlo, vhi, cbef.astype(jnp.float32), cnt.astype(jnp.float32)

        vloL, vhiL, cbefL, cntL = _locate(rL, cumL, T_LO)
        vloU, vhiU, cbefU, cntU = _locate(rU, cumU, T_UP)
        fz = jnp.float32(0.0)
        resv = fz
        for lane_ix, val in ((0, vloL), (1, vhiL), (2, cbefL), (3, cntL),
                             (4, vloU), (5, vhiU), (6, cbefU), (7, cntU)):
            resv = jnp.where(iota == lane_ix, val, resv)
        outv[0, :] = resv
        pltpu.sync_copy(outv, out_hbm.at[row])


_sc_quantile = pl.kernel(
    _sc_body,
    out_type=jax.ShapeDtypeStruct((B, 1, 16), jnp.float32),
    mesh=plsc.VectorSubcoreMesh(core_axis_name="c", subcore_axis_name="s"),
    compiler_params=pltpu.CompilerParams(needs_layout_passes=False),
    scratch_types=[
        pltpu.VMEM((65536,), jnp.int32),
        pltpu.VMEM((CROWS, 512), jnp.float32),
        pltpu.VMEM((CROWS, 512), jnp.float32),
        pltpu.VMEM((1, 16), jnp.float32),
        pltpu.SemaphoreType.DMA,
        pltpu.SemaphoreType.DMA,
    ],
)


def _norm_body(b_ref, x_ref, o_ref):
    def interp(r, vlo, vhi, cbef, cnt, pos):
        return vlo + (vhi - vlo) * ((jnp.float32(pos) - cbef
                                     + jnp.float32(0.5)) / cnt)

    for r in range(2):
        lo = interp(r, b_ref[r, 0, 0], b_ref[r, 0, 1], b_ref[r, 0, 2],
                    b_ref[r, 0, 3], POS_LO)
        up = interp(r, b_ref[r, 0, 4], b_ref[r, 0, 5], b_ref[r, 0, 6],
                    b_ref[r, 0, 7], POS_UP)
        xv = x_ref[r]
        o_ref[r] = (jnp.maximum(jnp.minimum(xv, up), lo) - lo) / (up - lo)


_tc_normalize = pl.pallas_call(
    _norm_body,
    grid=(B // 2,),
    in_specs=[
        pl.BlockSpec((2, 1, 16), lambda i: (i, 0, 0)),
        pl.BlockSpec((2, 1, 512, 512), lambda i: (i, 0, 0, 0)),
    ],
    out_specs=pl.BlockSpec((2, 1, 512, 512), lambda i: (i, 0, 0, 0)),
    out_shape=jax.ShapeDtypeStruct((B, 1, 512, 512), jnp.float32),
)


def kernel(x):
    xf = x.reshape(B, ROWS, 512)
    bounds = _sc_quantile(xf)
    return _tc_normalize(bounds, x)


# speedup vs baseline: 3.0978x; 1.0524x over previous
import jax
import jax.numpy as jnp
from jax import lax
from jax.experimental import pallas as pl
from jax.experimental.pallas import tpu as pltpu
from jax.experimental.pallas import tpu_sc as plsc

B = 64
N = 262144
ROWS = 512
NW = 32
ROWS_PER_W = B // NW
CROWS = 16
CHUNK = CROWS * 512
NCHUNK = N // CHUNK
NV = CHUNK // 16
UNROLL = 16

POS_LO = 2621.43
POS_UP = 259521.58
T_LO = 2621
T_UP = 259521

_INT_MIN_PY = -2147483648


def _sc_body(x_hbm, out_hbm, hist, buf0, buf1, outv, sem0, sem1):
    _INT_MIN = jnp.int32(_INT_MIN_PY)
    cid = lax.axis_index("c")
    sid = lax.axis_index("s")
    wid = sid * 2 + cid
    iota = lax.iota(jnp.int32, 16)
    zeros16 = jnp.zeros((16,), jnp.int32)
    ones16 = jnp.ones((16,), jnp.int32)

    def start_chunk(row, c, buf, sem):
        pltpu.make_async_copy(
            x_hbm.at[row, pl.ds(c * CROWS, CROWS)], buf, sem).start()

    def wait_chunk(row, c, buf, sem):
        pltpu.make_async_copy(
            x_hbm.at[row, pl.ds(c * CROWS, CROWS)], buf, sem).wait()

    def process(buf):
        def _hist_body(j, _):
            j0 = j * UNROLL
            vs = []
            for u in range(UNROLL):
                jj = j0 + u
                ri = lax.shift_right_logical(jj, 5)
                col = lax.shift_left(jnp.bitwise_and(jj, 31), 4)
                vs.append(buf[ri, pl.ds(col, 16)])
            addrs = []
            for v in vs:
                k32 = plsc.bitcast(v, jnp.int32)
                m = lax.shift_right_arithmetic(k32, 31)
                key = lax.bitwise_xor(k32, lax.bitwise_or(m, _INT_MIN))
                lo4 = jnp.bitwise_and(lax.shift_right_logical(key, 16), 15)
                addrs.append(jnp.bitwise_or(lax.shift_left(lo4, 12),
                                            lax.shift_right_logical(key, 20)))
            for addr in addrs:
                plsc.addupdate_scatter(hist, [addr], ones16)
            return 0

        lax.fori_loop(0, NV // UNROLL, _hist_body, 0)

    for rr in range(ROWS_PER_W):
        row = wid * ROWS_PER_W + rr

        def _zero_hist(i, _):
            base = i * 256
            for u in range(16):
                hist[pl.ds(base + u * 16, 16)] = zeros16
            return 0

        lax.fori_loop(0, 256, _zero_hist, 0)

        start_chunk(row, 0, buf0, sem0)
        def _chunk_body(k, _):
            c0 = k * 2
            wait_chunk(row, c0, buf0, sem0)
            start_chunk(row, c0 + 1, buf1, sem1)
            process(buf0)
            wait_chunk(row, c0 + 1, buf1, sem1)

            @pl.when(k < NCHUNK // 2 - 1)
            def _():
                start_chunk(row, c0 + 2, buf0, sem0)

            process(buf1)
            return 0

        lax.fori_loop(0, NCHUNK // 2, _chunk_body, 0)

        def _plane_sums(r):
            base = r * 16
            acc = hist[pl.ds(base, 16)]
            for l in range(1, 16):
                acc = acc + hist[pl.ds(l * 4096 + base, 16)]
            return acc

        def _cscan(r, carry):
            cum, rL, cumL, rU, cumU = carry
            tot = jnp.sum(_plane_sums(r))
            nxt = cum + tot
            condL = (cum <= T_LO) & (T_LO < nxt)
            condU = (cum <= T_UP) & (T_UP < nxt)
            rL = lax.select(condL, r, rL)
            cumL = lax.select(condL, cum, cumL)
            rU = lax.select(condU, r, rU)
            cumU = lax.select(condU, cum, cumU)
            return nxt, rL, cumL, rU, cumU

        z = jnp.int32(0)
        _, rL, cumL, rU, cumU = lax.fori_loop(0, 256, _cscan, (z, z, z, z, z))

        def _locate(r, cumbef, t):
            acc = _plane_sums(r)
            s = plsc.cumsum(acc)
            lane = jnp.sum((cumbef + s <= t).astype(jnp.int32))
            cbef_g = cumbef + jnp.sum(jnp.where(iota == lane, s - acc, 0))
            grp = r * 16 + lane
            vec = plsc.load_gather(hist, [iota * 4096 + grp])
            s2 = plsc.cumsum(vec)
            lane2 = jnp.sum((cbef_g + s2 <= t).astype(jnp.int32))
            cnt = jnp.sum(jnp.where(iota == lane2, vec, 0))
            cbef = cbef_g + jnp.sum(jnp.where(iota == lane2, s2 - vec, 0))
            fbin = grp * 16 + lane2
            klo = lax.shift_left(fbin, 16)
            khi = lax.shift_left(fbin + 1, 16)
            kv = jnp.where(iota == 0, klo, khi)
            bits = jnp.where(kv < 0,
                             lax.bitwise_xor(kv, _INT_MIN),
                             lax.bitwise_not(kv))
            fv = plsc.bitcast(bits, jnp.float32)
            neg_big = jnp.float32(-3.4e38)
            vlo = jnp.max(jnp.where(iota == 0, fv, neg_big))
            vhi = jnp.max(jnp.where(iota == 1, fv, neg_big))
            return vlo, vhi, cbef.astype(jnp.float32), cnt.astype(jnp.float32)

        vloL, vhiL, cbefL, cntL = _locate(rL, cumL, T_LO)
        vloU, vhiU, cbefU, cntU = _locate(rU, cumU, T_UP)
        fz = jnp.float32(0.0)
        resv = fz
        for lane_ix, val in ((0, vloL), (1, vhiL), (2, cbefL), (3, cntL),
                             (4, vloU), (5, vhiU), (6, cbefU), (7, cntU)):
            resv = jnp.where(iota == lane_ix, val, resv)
        outv[0, :] = resv
        pltpu.sync_copy(outv, out_hbm.at[row])


_sc_quantile = pl.kernel(
    _sc_body,
    out_type=jax.ShapeDtypeStruct((B, 1, 16), jnp.float32),
    mesh=plsc.VectorSubcoreMesh(core_axis_name="c", subcore_axis_name="s"),
    compiler_params=pltpu.CompilerParams(needs_layout_passes=False),
    scratch_types=[
        pltpu.VMEM((65536,), jnp.int32),
        pltpu.VMEM((CROWS, 512), jnp.float32),
        pltpu.VMEM((CROWS, 512), jnp.float32),
        pltpu.VMEM((1, 16), jnp.float32),
        pltpu.SemaphoreType.DMA,
        pltpu.SemaphoreType.DMA,
    ],
)


def _norm_body(b_ref, x_ref, o_ref):
    def interp(r, vlo, vhi, cbef, cnt, pos):
        return vlo + (vhi - vlo) * ((jnp.float32(pos) - cbef
                                     + jnp.float32(0.5)) / cnt)

    for r in range(4):
        lo = interp(r, b_ref[r, 0, 0], b_ref[r, 0, 1], b_ref[r, 0, 2],
                    b_ref[r, 0, 3], POS_LO)
        up = interp(r, b_ref[r, 0, 4], b_ref[r, 0, 5], b_ref[r, 0, 6],
                    b_ref[r, 0, 7], POS_UP)
        xv = x_ref[r]
        o_ref[r] = (jnp.maximum(jnp.minimum(xv, up), lo) - lo) / (up - lo)


_tc_normalize = pl.pallas_call(
    _norm_body,
    grid=(B // 4,),
    in_specs=[
        pl.BlockSpec((4, 1, 16), lambda i: (i, 0, 0)),
        pl.BlockSpec((4, 1, 512, 512), lambda i: (i, 0, 0, 0)),
    ],
    out_specs=pl.BlockSpec((4, 1, 512, 512), lambda i: (i, 0, 0, 0)),
    out_shape=jax.ShapeDtypeStruct((B, 1, 512, 512), jnp.float32),
)


def kernel(x):
    xf = x.reshape(B, ROWS, 512)
    bounds = _sc_quantile(xf)
    return _tc_normalize(bounds, x)


# speedup vs baseline: 3.1162x; 1.0059x over previous
import jax
import jax.numpy as jnp
from jax import lax
from jax.experimental import pallas as pl
from jax.experimental.pallas import tpu as pltpu
from jax.experimental.pallas import tpu_sc as plsc

B = 64
N = 262144
ROWS = 512
NW = 32
ROWS_PER_W = B // NW
CROWS = 16
CHUNK = CROWS * 512
NCHUNK = N // CHUNK
NV = CHUNK // 16
UNROLL = 32

POS_LO = 2621.43
POS_UP = 259521.58
T_LO = 2621
T_UP = 259521

_INT_MIN_PY = -2147483648


def _sc_body(x_hbm, out_hbm, hist, buf0, buf1, outv, sem0, sem1):
    _INT_MIN = jnp.int32(_INT_MIN_PY)
    cid = lax.axis_index("c")
    sid = lax.axis_index("s")
    wid = sid * 2 + cid
    iota = lax.iota(jnp.int32, 16)
    zeros16 = jnp.zeros((16,), jnp.int32)
    ones16 = jnp.ones((16,), jnp.int32)

    def start_chunk(row, c, buf, sem):
        pltpu.make_async_copy(
            x_hbm.at[row, pl.ds(c * CROWS, CROWS)], buf, sem).start()

    def wait_chunk(row, c, buf, sem):
        pltpu.make_async_copy(
            x_hbm.at[row, pl.ds(c * CROWS, CROWS)], buf, sem).wait()

    def process(buf):
        def _hist_body(j, _):
            j0 = j * UNROLL
            vs = []
            for u in range(UNROLL):
                jj = j0 + u
                ri = lax.shift_right_logical(jj, 5)
                col = lax.shift_left(jnp.bitwise_and(jj, 31), 4)
                vs.append(buf[ri, pl.ds(col, 16)])
            addrs = []
            for v in vs:
                k32 = plsc.bitcast(v, jnp.int32)
                m = lax.shift_right_arithmetic(k32, 31)
                key = lax.bitwise_xor(k32, lax.bitwise_or(m, _INT_MIN))
                lo4 = jnp.bitwise_and(lax.shift_right_logical(key, 16), 15)
                addrs.append(jnp.bitwise_or(lax.shift_left(lo4, 12),
                                            lax.shift_right_logical(key, 20)))
            for addr in addrs:
                plsc.addupdate_scatter(hist, [addr], ones16)
            return 0

        lax.fori_loop(0, NV // UNROLL, _hist_body, 0)

    for rr in range(ROWS_PER_W):
        row = wid * ROWS_PER_W + rr

        def _zero_hist(i, _):
            base = i * 256
            for u in range(16):
                hist[pl.ds(base + u * 16, 16)] = zeros16
            return 0

        lax.fori_loop(0, 256, _zero_hist, 0)

        start_chunk(row, 0, buf0, sem0)
        def _chunk_body(k, _):
            c0 = k * 2
            wait_chunk(row, c0, buf0, sem0)
            start_chunk(row, c0 + 1, buf1, sem1)
            process(buf0)
            wait_chunk(row, c0 + 1, buf1, sem1)

            @pl.when(k < NCHUNK // 2 - 1)
            def _():
                start_chunk(row, c0 + 2, buf0, sem0)

            process(buf1)
            return 0

        lax.fori_loop(0, NCHUNK // 2, _chunk_body, 0)

        def _plane_sums(r):
            base = r * 16
            acc = hist[pl.ds(base, 16)]
            for l in range(1, 16):
                acc = acc + hist[pl.ds(l * 4096 + base, 16)]
            return acc

        def _cscan(r, carry):
            cum, rL, cumL, rU, cumU = carry
            tot = jnp.sum(_plane_sums(r))
            nxt = cum + tot
            condL = (cum <= T_LO) & (T_LO < nxt)
            condU = (cum <= T_UP) & (T_UP < nxt)
            rL = lax.select(condL, r, rL)
            cumL = lax.select(condL, cum, cumL)
            rU = lax.select(condU, r, rU)
            cumU = lax.select(condU, cum, cumU)
            return nxt, rL, cumL, rU, cumU

        z = jnp.int32(0)
        _, rL, cumL, rU, cumU = lax.fori_loop(0, 256, _cscan, (z, z, z, z, z))

        def _locate(r, cumbef, t):
            acc = _plane_sums(r)
            s = plsc.cumsum(acc)
            lane = jnp.sum((cumbef + s <= t).astype(jnp.int32))
            cbef_g = cumbef + jnp.sum(jnp.where(iota == lane, s - acc, 0))
            grp = r * 16 + lane
            vec = plsc.load_gather(hist, [iota * 4096 + grp])
            s2 = plsc.cumsum(vec)
            lane2 = jnp.sum((cbef_g + s2 <= t).astype(jnp.int32))
            cnt = jnp.sum(jnp.where(iota == lane2, vec, 0))
            cbef = cbef_g + jnp.sum(jnp.where(iota == lane2, s2 - vec, 0))
            fbin = grp * 16 + lane2
            klo = lax.shift_left(fbin, 16)
            khi = lax.shift_left(fbin + 1, 16)
            kv = jnp.where(iota == 0, klo, khi)
            bits = jnp.where(kv < 0,
                             lax.bitwise_xor(kv, _INT_MIN),
                             lax.bitwise_not(kv))
            fv = plsc.bitcast(bits, jnp.float32)
            neg_big = jnp.float32(-3.4e38)
            vlo = jnp.max(jnp.where(iota == 0, fv, neg_big))
            vhi = jnp.max(jnp.where(iota == 1, fv, neg_big))
            return vlo, vhi, cbef.astype(jnp.float32), cnt.astype(jnp.float32)

        vloL, vhiL, cbefL, cntL = _locate(rL, cumL, T_LO)
        vloU, vhiU, cbefU, cntU = _locate(rU, cumU, T_UP)
        fz = jnp.float32(0.0)
        resv = fz
        for lane_ix, val in ((0, vloL), (1, vhiL), (2, cbefL), (3, cntL),
                             (4, vloU), (5, vhiU), (6, cbefU), (7, cntU)):
            resv = jnp.where(iota == lane_ix, val, resv)
        outv[0, :] = resv
        pltpu.sync_copy(outv, out_hbm.at[row])


_sc_quantile = pl.kernel(
    _sc_body,
    out_type=jax.ShapeDtypeStruct((B, 1, 16), jnp.float32),
    mesh=plsc.VectorSubcoreMesh(core_axis_name="c", subcore_axis_name="s"),
    compiler_params=pltpu.CompilerParams(needs_layout_passes=False),
    scratch_types=[
        pltpu.VMEM((65536,), jnp.int32),
        pltpu.VMEM((CROWS, 512), jnp.float32),
        pltpu.VMEM((CROWS, 512), jnp.float32),
        pltpu.VMEM((1, 16), jnp.float32),
        pltpu.SemaphoreType.DMA,
        pltpu.SemaphoreType.DMA,
    ],
)


def _norm_body(b_ref, x_ref, o_ref):
    def interp(r, vlo, vhi, cbef, cnt, pos):
        return vlo + (vhi - vlo) * ((jnp.float32(pos) - cbef
                                     + jnp.float32(0.5)) / cnt)

    for r in range(4):
        lo = interp(r, b_ref[r, 0, 0], b_ref[r, 0, 1], b_ref[r, 0, 2],
                    b_ref[r, 0, 3], POS_LO)
        up = interp(r, b_ref[r, 0, 4], b_ref[r, 0, 5], b_ref[r, 0, 6],
                    b_ref[r, 0, 7], POS_UP)
        xv = x_ref[r]
        o_ref[r] = (jnp.maximum(jnp.minimum(xv, up), lo) - lo) / (up - lo)


_tc_normalize = pl.pallas_call(
    _norm_body,
    grid=(B // 4,),
    in_specs=[
        pl.BlockSpec((4, 1, 16), lambda i: (i, 0, 0)),
        pl.BlockSpec((4, 1, 512, 512), lambda i: (i, 0, 0, 0)),
    ],
    out_specs=pl.BlockSpec((4, 1, 512, 512), lambda i: (i, 0, 0, 0)),
    out_shape=jax.ShapeDtypeStruct((B, 1, 512, 512), jnp.float32),
)


def kernel(x):
    xf = x.reshape(B, ROWS, 512)
    bounds = _sc_quantile(xf)
    return _tc_normalize(bounds, x)
